# baseline (speedup 1.0000x reference)
import jax
import jax.numpy as jnp
from jax import lax
from jax.experimental import pallas as pl
from jax.experimental.pallas import tpu as pltpu
from jax.experimental.pallas import tpu_sc as plsc

N = 2048
C_IN = 16
C_OUT = 64
NUM_CELLS = 16
L = 16
NW = 32
CPW = N // NW
ROWS = CPW * NUM_CELLS
UNROLL = 2
SENT_J = N
RADIUS = 0.1

T_OUT = 0.01
T_IN = 0.0025
M = 0.1001
NSLAB = 4


def _sc_body(px_hbm, py_hbm, pz_hbm, attr_hbm, ord_hbm, sum_hbm, cnt_hbm,
             px, py, pz, attr_v, acc, cnt, pairbuf,
             sx, sy, sz, sord, skey, lob, hib):
    wid = lax.axis_index("s") * 2 + lax.axis_index("c")
    pltpu.sync_copy(px_hbm, px)
    pltpu.sync_copy(py_hbm, py)
    pltpu.sync_copy(pz_hbm, pz)
    pltpu.sync_copy(attr_hbm, attr_v.at[pl.ds(0, N * C_IN)])
    pltpu.sync_copy(ord_hbm, sord.at[pl.ds(0, N)])

    lanes = lax.iota(jnp.int32, L)
    ones = jnp.ones((L,), jnp.float32)
    zeros = jnp.zeros((L,), jnp.float32)
    inf16 = jnp.full((L,), jnp.inf, jnp.float32)
    sentv = jnp.full((L,), SENT_J, jnp.int32)

    for off in (N, N + L):
        sx[pl.ds(off, L)] = inf16
        sy[pl.ds(off, L)] = inf16
        sz[pl.ds(off, L)] = inf16
        sord[pl.ds(off, L)] = sentv
    attr_v[pl.ds(N * C_IN, L)] = zeros

    def sort_body(k, _):
        ov = sord[pl.ds(k * L, L)]
        sxv = plsc.load_gather(px, [ov])
        syv = plsc.load_gather(py, [ov])
        sx[pl.ds(k * L, L)] = sxv
        sy[pl.ds(k * L, L)] = syv
        sz[pl.ds(k * L, L)] = plsc.load_gather(pz, [ov])
        skey[pl.ds(k * L, L)] = (sxv * 10.0).astype(jnp.int32).astype(
            jnp.float32) + syv
        return 0
    lax.fori_loop(0, N // L, sort_body, 0)

    def zero_body(r, _):
        for u in range(8):
            acc[pl.ds((r * 8 + u) * L, L)] = zeros
        return 0
    lax.fori_loop(0, (ROWS + 8) // 8, zero_body, 0)

    def zcnt_body(r, _):
        cnt[pl.ds(r * L, L)] = zeros
        return 0
    lax.fori_loop(0, (ROWS + L) // L, zcnt_body, 0)

    def bs_body(t, _):
        cxv = px[pl.ds(wid * CPW + t * L, L)]
        cyv = py[pl.ds(wid * CPW + t * L, L)]
        slov = (cxv - M) * 10.0
        slot = slov.astype(jnp.int32).astype(jnp.float32)
        slo = jnp.where(slot > slov, slot - 1.0, slot)
        ylo = jnp.maximum(cyv - M, 0.0)
        yhi = cyv + M
        for k in range(NSLAB):
            s = slo + k
            klo = s + ylo
            khi = jnp.minimum(s + yhi, s + 1.0)
            lo = jnp.zeros((L,), jnp.int32)
            hi = jnp.full((L,), N, jnp.int32)
            lo2 = jnp.zeros((L,), jnp.int32)
            hi2 = jnp.full((L,), N, jnp.int32)
            for _ in range(12):
                mid = (lo + hi) // 2
                v = plsc.load_gather(skey, [mid])
                c = v < klo
                lo = jnp.where(c, mid + 1, lo)
                hi = jnp.where(c, hi, mid)
                mid2 = (lo2 + hi2) // 2
                v2 = plsc.load_gather(skey, [mid2])
                c2 = v2 < khi
                lo2 = jnp.where(c2, mid2 + 1, lo2)
                hi2 = jnp.where(c2, hi2, mid2)
            lob[pl.ds(k * CPW + t * L, L)] = lo
            hib[pl.ds(k * CPW + t * L, L)] = lo2
        return 0
    lax.fori_loop(0, CPW // L, bs_body, 0)

    def center_body(ic, _):
        ci = wid * CPW + ic
        isel = jnp.full((L,), ci, jnp.int32)
        icel = jnp.full((L,), ic, jnp.int32)
        cx = plsc.load_gather(px, [isel])
        cy = plsc.load_gather(py, [isel])
        cz = plsc.load_gather(pz, [isel])
        arow_base = ic * NUM_CELLS

        carry = (0, jnp.zeros((L,), jnp.bool_), lanes)
        for k in range(NSLAB):
            kcel = icel + k * CPW
            lo = plsc.load_gather(lob, [kcel])[0]
            hi = plsc.load_gather(hib, [kcel])[0]
            nch = (hi - lo + L - 1) // L

            def scan_group(tt, carry, lo=lo, hi=hi):
                qp, vprev, jprev = carry
                for u in range(UNROLL):
                    j0 = lo + (tt * UNROLL + u) * L
                    jv = j0 + lanes
                    dx = sx[pl.ds(j0, L)] - cx
                    dy = sy[pl.ds(j0, L)] - cy
                    dz = sz[pl.ds(j0, L)] - cz
                    d2 = dx * dx + dy * dy + dz * dz
                    valid = (d2 < T_OUT) & (jv < hi)
                    plsc.store_compressed(pairbuf.at[pl.ds(qp, L)], jprev,
                                          mask=vprev)
                    npv = plsc.all_reduce_population_count(vprev)
                    qp = qp + npv[0]
                    vprev = valid
                    jprev = jv
                return (qp, vprev, jprev)
            carry = lax.fori_loop(0, (nch + UNROLL - 1) // UNROLL,
                                  scan_group, carry)
        qp, vlast, jlast = carry
        plsc.store_compressed(pairbuf.at[pl.ds(qp, L)], jlast, mask=vlast)
        npv = plsc.all_reduce_population_count(vlast)
        qp = qp + npv[0]

        pairbuf[pl.ds(qp, L)] = sentv
        nchd = (qp + L - 1) // L

        def drain_body(t, _):
            jv = pairbuf[pl.ds(t * L, L)]
            pxj = plsc.load_gather(sx, [jv])
            pyj = plsc.load_gather(sy, [jv])
            pzj = plsc.load_gather(sz, [jv])
            pv = plsc.load_gather(sord, [jv])
            dxv = pxj - cx
            dyv = pyj - cy
            dzv = pzj - cz
            d2v = dxv * dxv + dyv * dyv + dzv * dzv
            octv = ((dxv > 0).astype(jnp.int32) * 4
                    + (dyv > 0).astype(jnp.int32) * 2
                    + (dzv > 0).astype(jnp.int32))
            qv = jnp.where(d2v < T_IN, octv, octv + 8)
            rowv = jnp.where(jv < N, arow_base + qv, ROWS)
            plsc.addupdate_scatter(cnt, [rowv], ones)
            for l in range(L):
                a = rowv[l]
                p = pv[l]
                plsc.addupdate(acc.at[pl.ds(a * L, L)],
                               attr_v[pl.ds(p * L, L)])
            return 0
        lax.fori_loop(0, nchd, drain_body, 0)
        return 0
    lax.fori_loop(0, CPW, center_body, 0)

    pltpu.sync_copy(acc.at[pl.ds(0, ROWS * L)],
                    sum_hbm.at[pl.ds(wid * ROWS * L, ROWS * L)])
    pltpu.sync_copy(cnt.at[pl.ds(0, ROWS)],
                    cnt_hbm.at[pl.ds(wid * ROWS, ROWS)])


def _mm_body(s_ref, c_ref, w_ref, b_ref, o_ref):
    c = jnp.repeat(c_ref[:], C_IN, axis=1)
    g = s_ref[:] / jnp.maximum(c, 1.0)
    o_ref[:] = jnp.dot(g, w_ref[:],
                       preferred_element_type=jnp.float32) + b_ref[:]


def kernel(points, attributes, W, b):
    px_h = points[:, 0]
    py_h = points[:, 1]
    pz_h = points[:, 2]
    order = jnp.argsort(jnp.floor(px_h * 10.0) + py_h).astype(jnp.int32)
    attr_flat = attributes.reshape(N * C_IN)
    Wmat = jnp.transpose(W, (2, 1, 0)).reshape(NUM_CELLS * C_IN, C_OUT)
    b2d = b.reshape(1, C_OUT)

    mesh = plsc.VectorSubcoreMesh(core_axis_name="c", subcore_axis_name="s")
    sum_flat, cnt_flat = pl.kernel(
        _sc_body,
        out_type=(jax.ShapeDtypeStruct((N * NUM_CELLS * C_IN,), jnp.float32),
                  jax.ShapeDtypeStruct((N * NUM_CELLS,), jnp.float32)),
        mesh=mesh,
        compiler_params=pltpu.CompilerParams(needs_layout_passes=False),
        scratch_types=[
            pltpu.VMEM((N,), jnp.float32),
            pltpu.VMEM((N,), jnp.float32),
            pltpu.VMEM((N,), jnp.float32),
            pltpu.VMEM(((N + 1) * C_IN,), jnp.float32),
            pltpu.VMEM(((ROWS + 8) * C_IN,), jnp.float32),
            pltpu.VMEM((ROWS + L,), jnp.float32),
            pltpu.VMEM((N + 2 * L,), jnp.int32),
            pltpu.VMEM((N + 2 * L,), jnp.float32),
            pltpu.VMEM((N + 2 * L,), jnp.float32),
            pltpu.VMEM((N + 2 * L,), jnp.float32),
            pltpu.VMEM((N + 2 * L,), jnp.int32),
            pltpu.VMEM((N,), jnp.float32),
            pltpu.VMEM((NSLAB * CPW,), jnp.int32),
            pltpu.VMEM((NSLAB * CPW,), jnp.int32),
        ],
    )(px_h, py_h, pz_h, attr_flat, order)
    sums = sum_flat.reshape(N, NUM_CELLS * C_IN)
    cnts = cnt_flat.reshape(N, NUM_CELLS)

    out = pl.pallas_call(
        _mm_body,
        out_shape=jax.ShapeDtypeStruct((N, C_OUT), jnp.float32),
    )(sums, cnts, Wmat, b2d)
    return out


# speedup vs baseline: 1.1177x; 1.1177x over previous
import jax
import jax.numpy as jnp
from jax import lax
from jax.experimental import pallas as pl
from jax.experimental.pallas import tpu as pltpu
from jax.experimental.pallas import tpu_sc as plsc

N = 2048
C_IN = 16
C_OUT = 64
NUM_CELLS = 16
L = 16
NW = 32
CPW = N // NW
ROWS = CPW * NUM_CELLS
UNROLL = 2
SENT_J = N
RADIUS = 0.1

T_OUT = 0.01
T_IN = 0.0025
M = 0.1001
NSLAB = 4


def _sc_body(xyz_hbm, attr_hbm, ord_hbm, sum_hbm, cnt_hbm,
             pxyz, attr_v, acc, cnt, pairbuf,
             sx, sy, sz, sord, skey, lob, hib):
    wid = lax.axis_index("s") * 2 + lax.axis_index("c")
    pltpu.sync_copy(xyz_hbm, pxyz)
    pltpu.sync_copy(attr_hbm, attr_v.at[pl.ds(0, N * C_IN)])
    pltpu.sync_copy(ord_hbm, sord.at[pl.ds(0, N)])

    lanes = lax.iota(jnp.int32, L)
    ones = jnp.ones((L,), jnp.float32)
    zeros = jnp.zeros((L,), jnp.float32)
    inf16 = jnp.full((L,), jnp.inf, jnp.float32)
    sentv = jnp.full((L,), SENT_J, jnp.int32)

    for off in (N, N + L):
        sx[pl.ds(off, L)] = inf16
        sy[pl.ds(off, L)] = inf16
        sz[pl.ds(off, L)] = inf16
        sord[pl.ds(off, L)] = sentv
    attr_v[pl.ds(N * C_IN, L)] = zeros

    def sort_body(k, _):
        ov = sord[pl.ds(k * L, L)]
        sxv = plsc.load_gather(pxyz, [ov])
        syv = plsc.load_gather(pxyz, [ov + N])
        sx[pl.ds(k * L, L)] = sxv
        sy[pl.ds(k * L, L)] = syv
        sz[pl.ds(k * L, L)] = plsc.load_gather(pxyz, [ov + 2 * N])
        skey[pl.ds(k * L, L)] = (sxv * 10.0).astype(jnp.int32).astype(
            jnp.float32) + syv
        return 0
    lax.fori_loop(0, N // L, sort_body, 0)

    def zero_body(r, _):
        for u in range(8):
            acc[pl.ds((r * 8 + u) * L, L)] = zeros
            cnt[pl.ds((r * 8 + u) * L, L)] = zeros
        return 0
    lax.fori_loop(0, (ROWS + 8) // 8, zero_body, 0)

    def bs_body(t, _):
        cxv = pxyz[pl.ds(wid * CPW + t * L, L)]
        cyv = pxyz[pl.ds(N + wid * CPW + t * L, L)]
        slov = (cxv - M) * 10.0
        slot = slov.astype(jnp.int32).astype(jnp.float32)
        slo = jnp.where(slot > slov, slot - 1.0, slot)
        ylo = jnp.maximum(cyv - M, 0.0)
        yhi = cyv + M
        for k in range(NSLAB):
            s = slo + k
            klo = s + ylo
            khi = jnp.minimum(s + yhi, s + 1.0)
            lo = jnp.zeros((L,), jnp.int32)
            hi = jnp.full((L,), N, jnp.int32)
            lo2 = jnp.zeros((L,), jnp.int32)
            hi2 = jnp.full((L,), N, jnp.int32)
            for _ in range(12):
                mid = (lo + hi) // 2
                v = plsc.load_gather(skey, [mid])
                c = v < klo
                lo = jnp.where(c, mid + 1, lo)
                hi = jnp.where(c, hi, mid)
                mid2 = (lo2 + hi2) // 2
                v2 = plsc.load_gather(skey, [mid2])
                c2 = v2 < khi
                lo2 = jnp.where(c2, mid2 + 1, lo2)
                hi2 = jnp.where(c2, hi2, mid2)
            lob[pl.ds(k * CPW + t * L, L)] = lo
            hib[pl.ds(k * CPW + t * L, L)] = lo2
        return 0
    lax.fori_loop(0, CPW // L, bs_body, 0)

    def center_body(ic, _):
        ci = wid * CPW + ic
        isel = jnp.full((L,), ci, jnp.int32)
        icel = jnp.full((L,), ic, jnp.int32)
        cx = plsc.load_gather(pxyz, [isel])
        cy = plsc.load_gather(pxyz, [isel + N])
        cz = plsc.load_gather(pxyz, [isel + 2 * N])
        arow_base = ic * NUM_CELLS

        carry = (0, jnp.zeros((L,), jnp.bool_), lanes)
        for k in range(NSLAB):
            kcel = icel + k * CPW
            lo = plsc.load_gather(lob, [kcel])[0]
            hi = plsc.load_gather(hib, [kcel])[0]
            nch = (hi - lo + L - 1) // L

            def scan_group(tt, carry, lo=lo, hi=hi):
                qp, vprev, jprev = carry
                for u in range(UNROLL):
                    j0 = lo + (tt * UNROLL + u) * L
                    jv = j0 + lanes
                    dx = sx[pl.ds(j0, L)] - cx
                    dy = sy[pl.ds(j0, L)] - cy
                    dz = sz[pl.ds(j0, L)] - cz
                    d2 = dx * dx + dy * dy + dz * dz
                    valid = (d2 < T_OUT) & (jv < hi)
                    plsc.store_compressed(pairbuf.at[pl.ds(qp, L)], jprev,
                                          mask=vprev)
                    npv = plsc.all_reduce_population_count(vprev)
                    qp = qp + npv[0]
                    vprev = valid
                    jprev = jv
                return (qp, vprev, jprev)
            carry = lax.fori_loop(0, (nch + UNROLL - 1) // UNROLL,
                                  scan_group, carry)
        qp, vlast, jlast = carry
        plsc.store_compressed(pairbuf.at[pl.ds(qp, L)], jlast, mask=vlast)
        npv = plsc.all_reduce_population_count(vlast)
        qp = qp + npv[0]

        pairbuf[pl.ds(qp, L)] = sentv
        nchd = (qp + L - 1) // L

        def drain_body(t, _):
            jv = pairbuf[pl.ds(t * L, L)]
            pxj = plsc.load_gather(sx, [jv])
            pyj = plsc.load_gather(sy, [jv])
            pzj = plsc.load_gather(sz, [jv])
            pv = plsc.load_gather(sord, [jv])
            dxv = pxj - cx
            dyv = pyj - cy
            dzv = pzj - cz
            d2v = dxv * dxv + dyv * dyv + dzv * dzv
            octv = ((dxv > 0).astype(jnp.int32) * 4
                    + (dyv > 0).astype(jnp.int32) * 2
                    + (dzv > 0).astype(jnp.int32))
            qv = jnp.where(d2v < T_IN, octv, octv + 8)
            rowv = jnp.where(jv < N, arow_base + qv, ROWS)
            for l in range(L):
                a = rowv[l]
                p = pv[l]
                plsc.addupdate(acc.at[pl.ds(a * L, L)],
                               attr_v[pl.ds(p * L, L)])
                plsc.addupdate(cnt.at[pl.ds(a * L, L)], ones)
            return 0
        lax.fori_loop(0, nchd, drain_body, 0)
        return 0
    lax.fori_loop(0, CPW, center_body, 0)

    pltpu.sync_copy(acc.at[pl.ds(0, ROWS * L)],
                    sum_hbm.at[pl.ds(wid * ROWS * L, ROWS * L)])
    pltpu.sync_copy(cnt.at[pl.ds(0, ROWS * L)],
                    cnt_hbm.at[pl.ds(wid * ROWS * L, ROWS * L)])


def _mm_body(s_ref, c_ref, w_ref, b_ref, o_ref):
    g = s_ref[:] / jnp.maximum(c_ref[:], 1.0)
    o_ref[:] = jnp.dot(g, w_ref[:],
                       preferred_element_type=jnp.float32) + b_ref[:]


def kernel(points, attributes, W, b):
    px_h = points[:, 0]
    py_h = points[:, 1]
    pz_h = points[:, 2]
    order = jnp.argsort(jnp.floor(px_h * 10.0) + py_h).astype(jnp.int32)
    attr_flat = attributes.reshape(N * C_IN)
    Wmat = jnp.transpose(W, (2, 1, 0)).reshape(NUM_CELLS * C_IN, C_OUT)
    b2d = b.reshape(1, C_OUT)

    mesh = plsc.VectorSubcoreMesh(core_axis_name="c", subcore_axis_name="s")
    sum_flat, cnt_flat = pl.kernel(
        _sc_body,
        out_type=(jax.ShapeDtypeStruct((N * NUM_CELLS * C_IN,), jnp.float32),
                  jax.ShapeDtypeStruct((N * NUM_CELLS * C_IN,), jnp.float32)),
        mesh=mesh,
        compiler_params=pltpu.CompilerParams(needs_layout_passes=False),
        scratch_types=[
            pltpu.VMEM((3 * N,), jnp.float32),
            pltpu.VMEM(((N + 1) * C_IN,), jnp.float32),
            pltpu.VMEM(((ROWS + 8) * C_IN,), jnp.float32),
            pltpu.VMEM(((ROWS + 8) * C_IN,), jnp.float32),
            pltpu.VMEM((N + 2 * L,), jnp.int32),
            pltpu.VMEM((N + 2 * L,), jnp.float32),
            pltpu.VMEM((N + 2 * L,), jnp.float32),
            pltpu.VMEM((N + 2 * L,), jnp.float32),
            pltpu.VMEM((N + 2 * L,), jnp.int32),
            pltpu.VMEM((N,), jnp.float32),
            pltpu.VMEM((NSLAB * CPW,), jnp.int32),
            pltpu.VMEM((NSLAB * CPW,), jnp.int32),
        ],
    )(jnp.concatenate([px_h, py_h, pz_h]), attr_flat, order)
    sums = sum_flat.reshape(N, NUM_CELLS * C_IN)
    cnts = cnt_flat.reshape(N, NUM_CELLS * C_IN)

    out = pl.pallas_call(
        _mm_body,
        out_shape=jax.ShapeDtypeStruct((N, C_OUT), jnp.float32),
    )(sums, cnts, Wmat, b2d)
    return out


# speedup vs baseline: 1.1193x; 1.0015x over previous
import jax
import jax.numpy as jnp
from jax import lax
from jax.experimental import pallas as pl
from jax.experimental.pallas import tpu as pltpu
from jax.experimental.pallas import tpu_sc as plsc

N = 2048
C_IN = 16
C_OUT = 64
NUM_CELLS = 16
L = 16
NW = 32
CPW = N // NW
ROWS = CPW * NUM_CELLS
UNROLL = 2
SENT_J = N

T_OUT = 0.01
T_IN = 0.0025
M = 0.1001
NSLAB = 4


def _sc_body(xyz_hbm, attr_hbm, ord_hbm, sum_hbm, cnt_hbm,
             pxyz, attr_v, acc, cnt, pairbuf,
             sx, sy, sz, sord, skey, lob, hib):
    wid = lax.axis_index("s") * 2 + lax.axis_index("c")
    pltpu.sync_copy(xyz_hbm, pxyz)
    pltpu.sync_copy(attr_hbm, attr_v.at[pl.ds(0, N * C_IN)])
    pltpu.sync_copy(ord_hbm, sord.at[pl.ds(0, N)])

    lanes = lax.iota(jnp.int32, L)
    ones = jnp.ones((L,), jnp.float32)
    zeros = jnp.zeros((L,), jnp.float32)
    inf16 = jnp.full((L,), jnp.inf, jnp.float32)
    sentv = jnp.full((L,), SENT_J, jnp.int32)

    for off in (N, N + L):
        sx[pl.ds(off, L)] = inf16
        sy[pl.ds(off, L)] = inf16
        sz[pl.ds(off, L)] = inf16
        sord[pl.ds(off, L)] = sentv
    attr_v[pl.ds(N * C_IN, L)] = zeros

    def sort_body(k, _):
        ov = sord[pl.ds(k * L, L)]
        sxv = plsc.load_gather(pxyz, [ov])
        syv = plsc.load_gather(pxyz, [ov + N])
        sx[pl.ds(k * L, L)] = sxv
        sy[pl.ds(k * L, L)] = syv
        sz[pl.ds(k * L, L)] = plsc.load_gather(pxyz, [ov + 2 * N])
        skey[pl.ds(k * L, L)] = (sxv * 10.0).astype(jnp.int32).astype(
            jnp.float32) + syv
        return 0
    lax.fori_loop(0, N // L, sort_body, 0)

    def zero_body(r, _):
        for u in range(8):
            acc[pl.ds((r * 8 + u) * L, L)] = zeros
            cnt[pl.ds((r * 8 + u) * L, L)] = zeros
        return 0
    lax.fori_loop(0, (ROWS + 8) // 8, zero_body, 0)

    def bs_body(t, _):
        cxv = pxyz[pl.ds(wid * CPW + t * L, L)]
        cyv = pxyz[pl.ds(N + wid * CPW + t * L, L)]
        slov = (cxv - M) * 10.0
        slot = slov.astype(jnp.int32).astype(jnp.float32)
        slo = jnp.where(slot > slov, slot - 1.0, slot)
        ylo = jnp.maximum(cyv - M, 0.0)
        yhi = cyv + M
        for k in range(NSLAB):
            s = slo + k
            klo = s + ylo
            khi = jnp.minimum(s + yhi, s + 1.0)
            lo = jnp.zeros((L,), jnp.int32)
            hi = jnp.full((L,), N, jnp.int32)
            lo2 = jnp.zeros((L,), jnp.int32)
            hi2 = jnp.full((L,), N, jnp.int32)
            for _ in range(12):
                mid = (lo + hi) // 2
                v = plsc.load_gather(skey, [mid])
                c = v < klo
                lo = jnp.where(c, mid + 1, lo)
                hi = jnp.where(c, hi, mid)
                mid2 = (lo2 + hi2) // 2
                v2 = plsc.load_gather(skey, [mid2])
                c2 = v2 < khi
                lo2 = jnp.where(c2, mid2 + 1, lo2)
                hi2 = jnp.where(c2, hi2, mid2)
            lob[pl.ds(k * CPW + t * L, L)] = lo
            hib[pl.ds(k * CPW + t * L, L)] = lo2
        return 0
    lax.fori_loop(0, CPW // L, bs_body, 0)

    def center_body(ic, _):
        ci = wid * CPW + ic
        isel = jnp.full((L,), ci, jnp.int32)
        icel = jnp.full((L,), ic, jnp.int32)
        cx = plsc.load_gather(pxyz, [isel])
        cy = plsc.load_gather(pxyz, [isel + N])
        cz = plsc.load_gather(pxyz, [isel + 2 * N])
        arow_base = ic * NUM_CELLS

        carry = (0, jnp.zeros((L,), jnp.bool_), lanes)
        for k in range(NSLAB):
            kcel = icel + k * CPW
            lo = plsc.load_gather(lob, [kcel])[0]
            hi = plsc.load_gather(hib, [kcel])[0]
            nch = (hi - lo + L - 1) // L

            def scan_group(tt, carry, lo=lo, hi=hi):
                qp, vprev, jprev = carry
                for u in range(UNROLL):
                    j0 = lo + (tt * UNROLL + u) * L
                    jv = j0 + lanes
                    dx = sx[pl.ds(j0, L)] - cx
                    dy = sy[pl.ds(j0, L)] - cy
                    dz = sz[pl.ds(j0, L)] - cz
                    d2 = dx * dx + dy * dy + dz * dz
                    valid = (d2 < T_OUT) & (jv < hi)
                    plsc.store_compressed(pairbuf.at[pl.ds(qp, L)], jprev,
                                          mask=vprev)
                    npv = plsc.all_reduce_population_count(vprev)
                    qp = qp + npv[0]
                    vprev = valid
                    jprev = jv
                return (qp, vprev, jprev)
            carry = lax.fori_loop(0, (nch + UNROLL - 1) // UNROLL,
                                  scan_group, carry)
        qp, vlast, jlast = carry
        plsc.store_compressed(pairbuf.at[pl.ds(qp, L)], jlast, mask=vlast)
        npv = plsc.all_reduce_population_count(vlast)
        qp = qp + npv[0]

        pairbuf[pl.ds(qp, L)] = sentv
        nchd = (qp + L - 1) // L

        def drain_body(t, _):
            jv = pairbuf[pl.ds(t * L, L)]
            pxj = plsc.load_gather(sx, [jv])
            pyj = plsc.load_gather(sy, [jv])
            pzj = plsc.load_gather(sz, [jv])
            pv = plsc.load_gather(sord, [jv])
            dxv = pxj - cx
            dyv = pyj - cy
            dzv = pzj - cz
            d2v = dxv * dxv + dyv * dyv + dzv * dzv
            octv = ((dxv > 0).astype(jnp.int32) * 4
                    + (dyv > 0).astype(jnp.int32) * 2
                    + (dzv > 0).astype(jnp.int32))
            qv = jnp.where(d2v < T_IN, octv, octv + 8)
            rowv = jnp.where(jv < N, arow_base + qv, ROWS)
            for l in range(L):
                a = rowv[l]
                p = pv[l]
                plsc.addupdate(acc.at[pl.ds(a * L, L)],
                               attr_v[pl.ds(p * L, L)])
                plsc.addupdate(cnt.at[pl.ds(a * L, L)], ones)
            return 0
        lax.fori_loop(0, nchd, drain_body, 0)
        return 0
    lax.fori_loop(0, CPW, center_body, 0)

    pltpu.sync_copy(acc.at[pl.ds(0, ROWS * L)],
                    sum_hbm.at[pl.ds(wid * ROWS * L, ROWS * L)])
    pltpu.sync_copy(cnt.at[pl.ds(0, ROWS * L)],
                    cnt_hbm.at[pl.ds(wid * ROWS * L, ROWS * L)])


def _mm_body(s_ref, c_ref, w_ref, b_ref, o_ref):
    g = s_ref[:] / jnp.maximum(c_ref[:], 1.0)
    o_ref[:] = jnp.dot(g, w_ref[:],
                       preferred_element_type=jnp.float32) + b_ref[:]


def kernel(points, attributes, W, b):
    px_h = points[:, 0]
    py_h = points[:, 1]
    pz_h = points[:, 2]
    order = jnp.argsort(jnp.floor(px_h * 10.0) + py_h).astype(jnp.int32)
    attr_flat = attributes.reshape(N * C_IN)
    Wmat = jnp.transpose(W, (2, 1, 0)).reshape(NUM_CELLS * C_IN, C_OUT)
    b2d = b.reshape(1, C_OUT)

    mesh = plsc.VectorSubcoreMesh(core_axis_name="c", subcore_axis_name="s")
    sum_flat, cnt_flat = pl.kernel(
        _sc_body,
        out_type=(jax.ShapeDtypeStruct((N * NUM_CELLS * C_IN,), jnp.float32),
                  jax.ShapeDtypeStruct((N * NUM_CELLS * C_IN,), jnp.float32)),
        mesh=mesh,
        compiler_params=pltpu.CompilerParams(needs_layout_passes=False),
        scratch_types=[
            pltpu.VMEM((3 * N,), jnp.float32),
            pltpu.VMEM(((N + 1) * C_IN,), jnp.float32),
            pltpu.VMEM(((ROWS + 8) * C_IN,), jnp.float32),
            pltpu.VMEM(((ROWS + 8) * C_IN,), jnp.float32),
            pltpu.VMEM((N + 2 * L,), jnp.int32),
            pltpu.VMEM((N + 2 * L,), jnp.float32),
            pltpu.VMEM((N + 2 * L,), jnp.float32),
            pltpu.VMEM((N + 2 * L,), jnp.float32),
            pltpu.VMEM((N + 2 * L,), jnp.int32),
            pltpu.VMEM((N,), jnp.float32),
            pltpu.VMEM((NSLAB * CPW,), jnp.int32),
            pltpu.VMEM((NSLAB * CPW,), jnp.int32),
        ],
    )(jnp.concatenate([px_h, py_h, pz_h]), attr_flat, order)
    sums = sum_flat.reshape(N, NUM_CELLS * C_IN)
    cnts = cnt_flat.reshape(N, NUM_CELLS * C_IN)

    out = pl.pallas_call(
        _mm_body,
        out_shape=jax.ShapeDtypeStruct((N, C_OUT), jnp.float32),
    )(sums, cnts, Wmat, b2d)
    return out
